# Initial kernel scaffold; baseline (speedup 1.0000x reference)
import jax
import jax.numpy as jnp
from jax.experimental import pallas as pl


def kernel(input, index):
    raise NotImplementedError("write your pallas kernel here")


# speedup vs baseline: 4.3261x; 4.3261x over previous
import functools

import jax
import jax.numpy as jnp
from jax import lax
from jax.experimental import pallas as pl
from jax.experimental.pallas import tpu as pltpu
from jax.experimental.pallas import tpu_sc as plsc

NC = 2
NS = 16
L = 16
EPS = 1e-9

CH = 80
CR = 32


def _mesh():
    return plsc.VectorSubcoreMesh(
        core_axis_name="c", subcore_axis_name="s", num_cores=NC, num_subcores=NS
    )


def _make_phase_a(E, D, NP):
    ept = E // (NC * NS)
    n_chunks = ept // CH
    rpt = NP // NS

    @functools.partial(
        pl.kernel,
        out_type=[
            jax.ShapeDtypeStruct((NC, NP, D), jnp.float32),
            jax.ShapeDtypeStruct((NC, NP), jnp.float32),
        ],
        mesh=_mesh(),
        scratch_types=[
            pltpu.VMEM_SHARED((NP, D), jnp.float32),
            pltpu.VMEM_SHARED((NP,), jnp.float32),
            pltpu.VMEM((2, CH, D), jnp.float32),
            pltpu.VMEM((2, CH), jnp.int32),
            pltpu.VMEM((CH,), jnp.float32),
            pltpu.VMEM((CR, D), jnp.float32),
            pltpu.VMEM((rpt,), jnp.float32),
        ],
    )
    def phase_a(x_hbm, seg_hbm, acc_out, cnt_out, acc_sh, cnt_sh,
                rows_v, idx_v, ones_v, zrow_v, zcnt_v):
        c = lax.axis_index("c")
        s = lax.axis_index("s")
        g = c * NS + s

        zero = jnp.zeros((L,), jnp.float32)
        one = jnp.ones((L,), jnp.float32)
        for i in range(CH // L):
            ones_v[pl.ds(i * L, L)] = one
        for i in range(rpt // L):
            zcnt_v[pl.ds(i * L, L)] = zero

        def zrow_body(r, _):
            for k in range(D // L):
                zrow_v[r, pl.ds(k * L, L)] = zero
            return 0

        lax.fori_loop(0, CR, zrow_body, 0)

        def zinit_body(j, _):
            rb = s * rpt + j * CR
            pltpu.sync_copy(zrow_v, acc_sh.at[pl.ds(rb, CR), :])
            return 0

        lax.fori_loop(0, rpt // CR, zinit_body, 0)
        pltpu.sync_copy(zcnt_v, cnt_sh.at[pl.ds(s * rpt, rpt)])
        plsc.subcore_barrier()

        def edge_body(j, _):
            eb = g * ept + j * CH
            pltpu.sync_copy(seg_hbm.at[pl.ds(eb, CH)], idx_v.at[0])
            pltpu.sync_copy(x_hbm.at[pl.ds(eb, CH), :], rows_v.at[0])
            pltpu.sync_copy(rows_v.at[0], acc_sh.at[idx_v.at[0]], add=True)
            pltpu.sync_copy(ones_v, cnt_sh.at[idx_v.at[0]], add=True)
            return 0

        lax.fori_loop(0, n_chunks, edge_body, 0)
        plsc.subcore_barrier()

        rb = s * rpt
        pltpu.sync_copy(acc_sh.at[pl.ds(rb, rpt), :], acc_out.at[c, pl.ds(rb, rpt), :])
        pltpu.sync_copy(cnt_sh.at[pl.ds(rb, rpt)], cnt_out.at[c, pl.ds(rb, rpt)])

    return phase_a


def _make_phase_b(E, D, NP):
    ept = E // (NC * NS)
    n_chunks = ept // CH
    rpt = NP // NS

    @functools.partial(
        pl.kernel,
        out_type=jax.ShapeDtypeStruct((E, D), jnp.float32),
        mesh=_mesh(),
        scratch_types=[
            pltpu.VMEM_SHARED((NP, D), jnp.float32),
            pltpu.VMEM((CR, D), jnp.float32),
            pltpu.VMEM((CR, D), jnp.float32),
            pltpu.VMEM((CR,), jnp.float32),
            pltpu.VMEM((CR,), jnp.float32),
            pltpu.VMEM((CR, D), jnp.float32),
            pltpu.VMEM((2, CH), jnp.int32),
            pltpu.VMEM((2, CH, D), jnp.float32),
        ],
    )
    def phase_b(acc_hbm, cnt_hbm, seg_hbm, out_hbm, table_sh,
                a0_v, a1_v, c0_v, c1_v, pool_v, idx_v, rows_v):
        c = lax.axis_index("c")
        s = lax.axis_index("s")
        g = c * NS + s

        def combine_body(j, _):
            rb = s * rpt + j * CR
            pltpu.sync_copy(acc_hbm.at[0, pl.ds(rb, CR), :], a0_v)
            pltpu.sync_copy(acc_hbm.at[1, pl.ds(rb, CR), :], a1_v)
            pltpu.sync_copy(cnt_hbm.at[0, pl.ds(rb, CR)], c0_v)
            pltpu.sync_copy(cnt_hbm.at[1, pl.ds(rb, CR)], c1_v)

            def grp_body(gi, _):
                base = gi * L
                gsl = pl.ds(base, L)
                rcp16 = 1.0 / (c0_v[gsl] + c1_v[gsl] + jnp.float32(EPS))
                for j in range(L):
                    rcp = rcp16[j]
                    r = base + j
                    for k in range(D // L):
                        sl = pl.ds(k * L, L)
                        pool_v[r, sl] = (a0_v[r, sl] + a1_v[r, sl]) * rcp
                return 0

            lax.fori_loop(0, CR // L, grp_body, 0)
            pltpu.sync_copy(pool_v, table_sh.at[pl.ds(rb, CR), :])
            return 0

        lax.fori_loop(0, rpt // CR, combine_body, 0)
        plsc.subcore_barrier()

        def edge_body(j, _):
            eb = g * ept + j * CH
            pltpu.sync_copy(seg_hbm.at[pl.ds(eb, CH)], idx_v.at[0])
            pltpu.sync_copy(table_sh.at[idx_v.at[0]], rows_v.at[0])
            pltpu.sync_copy(rows_v.at[0], out_hbm.at[pl.ds(eb, CH), :])
            return 0

        lax.fori_loop(0, n_chunks, edge_body, 0)

    return phase_b


def kernel(input, index):
    E, D = input.shape
    NP = ((10000 + NS * CR - 1) // (NS * CR)) * (NS * CR)
    seg = index[:, 1]
    acc, cnt = _make_phase_a(E, D, NP)(input, seg)
    return _make_phase_b(E, D, NP)(acc, cnt, seg)


# speedup vs baseline: 6.8312x; 1.5790x over previous
import functools

import jax
import jax.numpy as jnp
from jax import lax
from jax.experimental import pallas as pl
from jax.experimental.pallas import tpu as pltpu
from jax.experimental.pallas import tpu_sc as plsc

NC = 2
NS = 16
L = 16
EPS = 1e-9

CH = 80
CR = 32


def _mesh():
    return plsc.VectorSubcoreMesh(
        core_axis_name="c", subcore_axis_name="s", num_cores=NC, num_subcores=NS
    )


def _make_phase_a(E, D, NP):
    ept = E // (NC * NS)
    n_chunks = ept // CH
    n_pairs = n_chunks // 2
    rpt = NP // NS

    @functools.partial(
        pl.kernel,
        out_type=[
            jax.ShapeDtypeStruct((NC, NP, D), jnp.float32),
            jax.ShapeDtypeStruct((NC, NP), jnp.float32),
        ],
        mesh=_mesh(),
        scratch_types=[
            pltpu.VMEM_SHARED((NP, D), jnp.float32),
            pltpu.VMEM_SHARED((NP,), jnp.float32),
            pltpu.VMEM((2, CH, D), jnp.float32),
            pltpu.VMEM((2, CH), jnp.int32),
            pltpu.VMEM((CH,), jnp.float32),
            pltpu.VMEM((CR, D), jnp.float32),
            pltpu.VMEM((rpt,), jnp.float32),
            pltpu.SemaphoreType.DMA((2,)),
            pltpu.SemaphoreType.DMA((2,)),
            pltpu.SemaphoreType.DMA((2,)),
            pltpu.SemaphoreType.DMA((2,)),
        ],
    )
    def phase_a(x_hbm, seg_hbm, acc_out, cnt_out, acc_sh, cnt_sh,
                rows_v, idx_v, ones_v, zrow_v, zcnt_v,
                sem_lr, sem_li, sem_sa, sem_sc):
        c = lax.axis_index("c")
        s = lax.axis_index("s")
        g = c * NS + s

        zero = jnp.zeros((L,), jnp.float32)
        one = jnp.ones((L,), jnp.float32)
        for i in range(CH // L):
            ones_v[pl.ds(i * L, L)] = one
        for i in range(rpt // L):
            zcnt_v[pl.ds(i * L, L)] = zero

        def zrow_body(r, _):
            for k in range(D // L):
                zrow_v[r, pl.ds(k * L, L)] = zero
            return 0

        lax.fori_loop(0, CR, zrow_body, 0)

        def zinit_body(j, _):
            rb = s * rpt + j * CR
            pltpu.sync_copy(zrow_v, acc_sh.at[pl.ds(rb, CR), :])
            return 0

        lax.fori_loop(0, rpt // CR, zinit_body, 0)
        pltpu.sync_copy(zcnt_v, cnt_sh.at[pl.ds(s * rpt, rpt)])
        plsc.subcore_barrier()

        def chunk(j, b, drain):
            if drain:
                pltpu.make_async_copy(
                    rows_v.at[b], acc_sh.at[idx_v.at[b]], sem_sa.at[b]).wait()
                pltpu.make_async_copy(
                    ones_v, cnt_sh.at[idx_v.at[b]], sem_sc.at[b]).wait()
            eb = g * ept + j * CH
            ld_i = pltpu.async_copy(
                seg_hbm.at[pl.ds(eb, CH)], idx_v.at[b], sem_li.at[b])
            ld_r = pltpu.async_copy(
                x_hbm.at[pl.ds(eb, CH), :], rows_v.at[b], sem_lr.at[b])
            ld_i.wait()
            ld_r.wait()
            pltpu.async_copy(
                rows_v.at[b], acc_sh.at[idx_v.at[b]], sem_sa.at[b], add=True)
            pltpu.async_copy(
                ones_v, cnt_sh.at[idx_v.at[b]], sem_sc.at[b], add=True)

        chunk(0, 0, False)
        chunk(1, 1, False)

        def pair_body(p, _):
            chunk(2 * p, 0, True)
            chunk(2 * p + 1, 1, True)
            return 0

        lax.fori_loop(1, n_pairs, pair_body, 0)
        if n_chunks % 2 == 1:
            chunk(n_chunks - 1, 0, True)
        for b in range(2):
            pltpu.make_async_copy(
                rows_v.at[b], acc_sh.at[idx_v.at[b]], sem_sa.at[b]).wait()
            pltpu.make_async_copy(
                ones_v, cnt_sh.at[idx_v.at[b]], sem_sc.at[b]).wait()
        plsc.subcore_barrier()

        rb = s * rpt
        pltpu.sync_copy(acc_sh.at[pl.ds(rb, rpt), :], acc_out.at[c, pl.ds(rb, rpt), :])
        pltpu.sync_copy(cnt_sh.at[pl.ds(rb, rpt)], cnt_out.at[c, pl.ds(rb, rpt)])

    return phase_a


def _make_phase_b(E, D, NP):
    ept = E // (NC * NS)
    n_chunks = ept // CH
    n_pairs = n_chunks // 2
    rpt = NP // NS

    @functools.partial(
        pl.kernel,
        out_type=jax.ShapeDtypeStruct((E, D), jnp.float32),
        mesh=_mesh(),
        scratch_types=[
            pltpu.VMEM_SHARED((NP, D), jnp.float32),
            pltpu.VMEM((CR, D), jnp.float32),
            pltpu.VMEM((CR, D), jnp.float32),
            pltpu.VMEM((CR,), jnp.float32),
            pltpu.VMEM((CR,), jnp.float32),
            pltpu.VMEM((CR, D), jnp.float32),
            pltpu.VMEM((2, CH), jnp.int32),
            pltpu.VMEM((2, CH, D), jnp.float32),
            pltpu.SemaphoreType.DMA((4,)),
            pltpu.SemaphoreType.DMA((2,)),
            pltpu.SemaphoreType.DMA((2,)),
        ],
    )
    def phase_b(acc_hbm, cnt_hbm, seg_hbm, out_hbm, table_sh,
                a0_v, a1_v, c0_v, c1_v, pool_v, idx_v, rows_v,
                sem_cl, sem_li, sem_w):
        c = lax.axis_index("c")
        s = lax.axis_index("s")
        g = c * NS + s

        def combine_body(j, _):
            rb = s * rpt + j * CR
            lds = [
                pltpu.async_copy(acc_hbm.at[0, pl.ds(rb, CR), :], a0_v, sem_cl.at[0]),
                pltpu.async_copy(acc_hbm.at[1, pl.ds(rb, CR), :], a1_v, sem_cl.at[1]),
                pltpu.async_copy(cnt_hbm.at[0, pl.ds(rb, CR)], c0_v, sem_cl.at[2]),
                pltpu.async_copy(cnt_hbm.at[1, pl.ds(rb, CR)], c1_v, sem_cl.at[3]),
            ]
            for ld in lds:
                ld.wait()

            def grp_body(gi, _):
                base = gi * L
                gsl = pl.ds(base, L)
                rcp16 = 1.0 / (c0_v[gsl] + c1_v[gsl] + jnp.float32(EPS))
                for jj in range(L):
                    rcp = rcp16[jj]
                    r = base + jj
                    for k in range(D // L):
                        sl = pl.ds(k * L, L)
                        pool_v[r, sl] = (a0_v[r, sl] + a1_v[r, sl]) * rcp
                return 0

            lax.fori_loop(0, CR // L, grp_body, 0)
            pltpu.sync_copy(pool_v, table_sh.at[pl.ds(rb, CR), :])
            return 0

        lax.fori_loop(0, rpt // CR, combine_body, 0)
        plsc.subcore_barrier()

        def chunk(j, b, drain):
            if drain:
                pltpu.make_async_copy(
                    rows_v.at[b], out_hbm.at[pl.ds(0, CH), :], sem_w.at[b]).wait()
            eb = g * ept + j * CH
            pltpu.async_copy(
                seg_hbm.at[pl.ds(eb, CH)], idx_v.at[b], sem_li.at[b]).wait()
            pltpu.sync_copy(table_sh.at[idx_v.at[b]], rows_v.at[b])
            pltpu.async_copy(
                rows_v.at[b], out_hbm.at[pl.ds(eb, CH), :], sem_w.at[b])

        chunk(0, 0, False)
        chunk(1, 1, False)

        def pair_body(p, _):
            chunk(2 * p, 0, True)
            chunk(2 * p + 1, 1, True)
            return 0

        lax.fori_loop(1, n_pairs, pair_body, 0)
        if n_chunks % 2 == 1:
            chunk(n_chunks - 1, 0, True)
        for b in range(2):
            pltpu.make_async_copy(
                rows_v.at[b], out_hbm.at[pl.ds(0, CH), :], sem_w.at[b]).wait()

    return phase_b


def kernel(input, index):
    E, D = input.shape
    NP = ((10000 + NS * CR - 1) // (NS * CR)) * (NS * CR)
    seg = index[:, 1]
    acc, cnt = _make_phase_a(E, D, NP)(input, seg)
    return _make_phase_b(E, D, NP)(acc, cnt, seg)


# speedup vs baseline: 7.6679x; 1.1225x over previous
import functools

import jax
import jax.numpy as jnp
from jax import lax
from jax.experimental import pallas as pl
from jax.experimental.pallas import tpu as pltpu
from jax.experimental.pallas import tpu_sc as plsc

NC = 2
NS = 16
L = 16
EPS = 1e-9

CH = 80
CR = 16


def _mesh():
    return plsc.VectorSubcoreMesh(
        core_axis_name="c", subcore_axis_name="s", num_cores=NC, num_subcores=NS
    )


def _make_phase_a(E, D, NP):
    ept = E // (NC * NS)
    n_chunks = ept // CH
    rpt = NP // NS

    @functools.partial(
        pl.kernel,
        out_type=[
            jax.ShapeDtypeStruct((NC, NP, D), jnp.float32),
            jax.ShapeDtypeStruct((NC, NP), jnp.float32),
        ],
        mesh=_mesh(),
        scratch_types=[
            pltpu.VMEM_SHARED((NP, D), jnp.float32),
            pltpu.VMEM_SHARED((NP,), jnp.float32),
            pltpu.VMEM((2, CH, D), jnp.float32),
            pltpu.VMEM((n_chunks, CH), jnp.int32),
            pltpu.VMEM((CH,), jnp.float32),
            pltpu.VMEM((CR, D), jnp.float32),
            pltpu.VMEM((rpt,), jnp.float32),
            pltpu.SemaphoreType.DMA,
            pltpu.SemaphoreType.DMA((2,)),
            pltpu.SemaphoreType.DMA((2,)),
            pltpu.SemaphoreType.DMA((2,)),
        ],
    )
    def phase_a(x_hbm, seg_hbm, acc_out, cnt_out, acc_sh, cnt_sh,
                rows_v, idx_v, ones_v, zrow_v, zcnt_v,
                sem_ix, sem_lr, sem_sa, sem_sc):
        c = lax.axis_index("c")
        s = lax.axis_index("s")
        g = c * NS + s

        idx_ld = pltpu.async_copy(seg_hbm.at[g], idx_v, sem_ix)

        zero = jnp.zeros((L,), jnp.float32)
        one = jnp.ones((L,), jnp.float32)
        for i in range(CH // L):
            ones_v[pl.ds(i * L, L)] = one
        for i in range(rpt // L):
            zcnt_v[pl.ds(i * L, L)] = zero

        def zrow_body(r, _):
            for k in range(D // L):
                zrow_v[r, pl.ds(k * L, L)] = zero
            return 0

        lax.fori_loop(0, CR, zrow_body, 0)

        def zinit_body(j, _):
            rb = s * rpt + j * CR
            pltpu.sync_copy(zrow_v, acc_sh.at[pl.ds(rb, CR), :])
            return 0

        lax.fori_loop(0, rpt // CR, zinit_body, 0)
        pltpu.sync_copy(zcnt_v, cnt_sh.at[pl.ds(s * rpt, rpt)])
        idx_ld.wait()
        plsc.subcore_barrier()

        def chunk(j, b, jd):
            if jd is not None:
                pltpu.make_async_copy(
                    rows_v.at[b], acc_sh.at[idx_v.at[jd]], sem_sa.at[b]).wait()
                pltpu.make_async_copy(
                    ones_v, cnt_sh.at[idx_v.at[jd]], sem_sc.at[b]).wait()
            eb = g * ept + j * CH
            pltpu.async_copy(
                x_hbm.at[pl.ds(eb, CH), :], rows_v.at[b], sem_lr.at[b]).wait()
            pltpu.async_copy(
                rows_v.at[b], acc_sh.at[idx_v.at[j]], sem_sa.at[b], add=True)
            pltpu.async_copy(
                ones_v, cnt_sh.at[idx_v.at[j]], sem_sc.at[b], add=True)

        chunk(0, 0, None)
        chunk(1, 1, None)

        def pair_body(p, _):
            j = 2 * p
            chunk(j, 0, j - 2)
            chunk(j + 1, 1, j - 1)
            return 0

        lax.fori_loop(1, n_chunks // 2, pair_body, 0)
        assert n_chunks % 2 == 1
        chunk(n_chunks - 1, 0, n_chunks - 3)
        for b, jd in ((0, n_chunks - 1), (1, n_chunks - 2)):
            pltpu.make_async_copy(
                rows_v.at[b], acc_sh.at[idx_v.at[jd]], sem_sa.at[b]).wait()
            pltpu.make_async_copy(
                ones_v, cnt_sh.at[idx_v.at[jd]], sem_sc.at[b]).wait()
        plsc.subcore_barrier()

        rb = s * rpt
        pltpu.sync_copy(acc_sh.at[pl.ds(rb, rpt), :], acc_out.at[c, pl.ds(rb, rpt), :])
        pltpu.sync_copy(cnt_sh.at[pl.ds(rb, rpt)], cnt_out.at[c, pl.ds(rb, rpt)])

    return phase_a


def _make_phase_b(E, D, NP):
    ept = E // (NC * NS)
    n_chunks = ept // CH
    rpt = NP // NS

    @functools.partial(
        pl.kernel,
        out_type=jax.ShapeDtypeStruct((E, D), jnp.float32),
        mesh=_mesh(),
        scratch_types=[
            pltpu.VMEM_SHARED((NP, D), jnp.float32),
            pltpu.VMEM((CR, D), jnp.float32),
            pltpu.VMEM((CR, D), jnp.float32),
            pltpu.VMEM((CR,), jnp.float32),
            pltpu.VMEM((CR,), jnp.float32),
            pltpu.VMEM((CR, D), jnp.float32),
            pltpu.VMEM((n_chunks, CH), jnp.int32),
            pltpu.VMEM((2, CH, D), jnp.float32),
            pltpu.SemaphoreType.DMA,
            pltpu.SemaphoreType.DMA((4,)),
            pltpu.SemaphoreType.DMA((2,)),
        ],
    )
    def phase_b(acc_hbm, cnt_hbm, seg_hbm, out_hbm, table_sh,
                a0_v, a1_v, c0_v, c1_v, pool_v, idx_v, rows_v,
                sem_ix, sem_cl, sem_w):
        c = lax.axis_index("c")
        s = lax.axis_index("s")
        g = c * NS + s

        idx_ld = pltpu.async_copy(seg_hbm.at[g], idx_v, sem_ix)

        def combine_body(j, _):
            rb = s * rpt + j * CR
            lds = [
                pltpu.async_copy(acc_hbm.at[0, pl.ds(rb, CR), :], a0_v, sem_cl.at[0]),
                pltpu.async_copy(acc_hbm.at[1, pl.ds(rb, CR), :], a1_v, sem_cl.at[1]),
                pltpu.async_copy(cnt_hbm.at[0, pl.ds(rb, CR)], c0_v, sem_cl.at[2]),
                pltpu.async_copy(cnt_hbm.at[1, pl.ds(rb, CR)], c1_v, sem_cl.at[3]),
            ]
            for ld in lds:
                ld.wait()

            def grp_body(gi, _):
                base = gi * L
                gsl = pl.ds(base, L)
                rcp16 = 1.0 / (c0_v[gsl] + c1_v[gsl] + jnp.float32(EPS))
                for jj in range(L):
                    rcp = rcp16[jj]
                    r = base + jj
                    for k in range(D // L):
                        sl = pl.ds(k * L, L)
                        pool_v[r, sl] = (a0_v[r, sl] + a1_v[r, sl]) * rcp
                return 0

            lax.fori_loop(0, CR // L, grp_body, 0)
            pltpu.sync_copy(pool_v, table_sh.at[pl.ds(rb, CR), :])
            return 0

        lax.fori_loop(0, rpt // CR, combine_body, 0)
        idx_ld.wait()
        plsc.subcore_barrier()

        def chunk(j, b, drain):
            if drain:
                pltpu.make_async_copy(
                    rows_v.at[b], out_hbm.at[pl.ds(0, CH), :], sem_w.at[b]).wait()
            eb = g * ept + j * CH
            pltpu.sync_copy(table_sh.at[idx_v.at[j]], rows_v.at[b])
            pltpu.async_copy(
                rows_v.at[b], out_hbm.at[pl.ds(eb, CH), :], sem_w.at[b])

        chunk(0, 0, False)
        chunk(1, 1, False)

        def pair_body(p, _):
            chunk(2 * p, 0, True)
            chunk(2 * p + 1, 1, True)
            return 0

        lax.fori_loop(1, n_chunks // 2, pair_body, 0)
        if n_chunks % 2 == 1:
            chunk(n_chunks - 1, 0, True)
        for b in range(2):
            pltpu.make_async_copy(
                rows_v.at[b], out_hbm.at[pl.ds(0, CH), :], sem_w.at[b]).wait()

    return phase_b


def kernel(input, index):
    E, D = input.shape
    NP = ((10000 + NS * CR - 1) // (NS * CR)) * (NS * CR)
    seg = index[:, 1].reshape(NC * NS, (E // CH) // (NC * NS), CH)
    acc, cnt = _make_phase_a(E, D, NP)(input, seg)
    return _make_phase_b(E, D, NP)(acc, cnt, seg)


# speedup vs baseline: 7.9867x; 1.0416x over previous
import functools

import jax
import jax.numpy as jnp
from jax import lax
from jax.experimental import pallas as pl
from jax.experimental.pallas import tpu as pltpu
from jax.experimental.pallas import tpu_sc as plsc

NC = 2
NS = 16
L = 16
EPS = 1e-9

CH = 80
CR = 16
CZ = 16


def _mesh():
    return plsc.VectorSubcoreMesh(
        core_axis_name="c", subcore_axis_name="s", num_cores=NC, num_subcores=NS
    )


def _make_phase_a(E, D, NP):
    ept = E // (NC * NS)
    n_chunks = ept // CH
    rpt = NP // NS

    @functools.partial(
        pl.kernel,
        out_type=[
            jax.ShapeDtypeStruct((NC, NP, D), jnp.float32),
            jax.ShapeDtypeStruct((NC, NP), jnp.float32),
        ],
        mesh=_mesh(),
        scratch_types=[
            pltpu.VMEM_SHARED((NP, D), jnp.float32),
            pltpu.VMEM_SHARED((NP,), jnp.float32),
            pltpu.VMEM((2, CH, D), jnp.float32),
            pltpu.VMEM((n_chunks, CH), jnp.int32),
            pltpu.VMEM((CH,), jnp.float32),
            pltpu.VMEM((CZ, D), jnp.float32),
            pltpu.VMEM((rpt,), jnp.float32),
            pltpu.SemaphoreType.DMA,
            pltpu.SemaphoreType.DMA((2,)),
            pltpu.SemaphoreType.DMA((2,)),
            pltpu.SemaphoreType.DMA((2,)),
        ],
    )
    def phase_a(x_hbm, seg_hbm, acc_out, cnt_out, acc_sh, cnt_sh,
                rows_v, idx_v, ones_v, zrow_v, zcnt_v,
                sem_ix, sem_lr, sem_sa, sem_sc):
        c = lax.axis_index("c")
        s = lax.axis_index("s")
        g = c * NS + s

        idx_ld = pltpu.async_copy(seg_hbm.at[g], idx_v, sem_ix)

        zero = jnp.zeros((L,), jnp.float32)
        one = jnp.ones((L,), jnp.float32)
        for i in range(CH // L):
            ones_v[pl.ds(i * L, L)] = one
        for i in range(rpt // L):
            zcnt_v[pl.ds(i * L, L)] = zero

        def zrow_body(r, _):
            for k in range(D // L):
                zrow_v[r, pl.ds(k * L, L)] = zero
            return 0

        lax.fori_loop(0, CZ, zrow_body, 0)

        def zinit_body(j, _):
            rb = s * rpt + j * CZ
            pltpu.sync_copy(zrow_v, acc_sh.at[pl.ds(rb, CZ), :])
            return 0

        lax.fori_loop(0, rpt // CZ, zinit_body, 0)
        pltpu.sync_copy(zcnt_v, cnt_sh.at[pl.ds(s * rpt, rpt)])
        idx_ld.wait()
        plsc.subcore_barrier()

        def chunk(j, b, jd):
            if jd is not None:
                pltpu.make_async_copy(
                    rows_v.at[b], acc_sh.at[idx_v.at[jd]], sem_sa.at[b]).wait()
                pltpu.make_async_copy(
                    ones_v, cnt_sh.at[idx_v.at[jd]], sem_sc.at[b]).wait()
            eb = g * ept + j * CH
            pltpu.async_copy(
                x_hbm.at[pl.ds(eb, CH), :], rows_v.at[b], sem_lr.at[b]).wait()
            pltpu.async_copy(
                rows_v.at[b], acc_sh.at[idx_v.at[j]], sem_sa.at[b], add=True)
            pltpu.async_copy(
                ones_v, cnt_sh.at[idx_v.at[j]], sem_sc.at[b], add=True)

        chunk(0, 0, None)
        chunk(1, 1, None)

        def pair_body(p, _):
            j = 2 * p
            chunk(j, 0, j - 2)
            chunk(j + 1, 1, j - 1)
            return 0

        lax.fori_loop(1, n_chunks // 2, pair_body, 0)
        assert n_chunks % 2 == 1
        chunk(n_chunks - 1, 0, n_chunks - 3)
        for b, jd in ((0, n_chunks - 1), (1, n_chunks - 2)):
            pltpu.make_async_copy(
                rows_v.at[b], acc_sh.at[idx_v.at[jd]], sem_sa.at[b]).wait()
            pltpu.make_async_copy(
                ones_v, cnt_sh.at[idx_v.at[jd]], sem_sc.at[b]).wait()
        plsc.subcore_barrier()

        rb = s * rpt
        pltpu.sync_copy(acc_sh.at[pl.ds(rb, rpt), :], acc_out.at[c, pl.ds(rb, rpt), :])
        pltpu.sync_copy(cnt_sh.at[pl.ds(rb, rpt)], cnt_out.at[c, pl.ds(rb, rpt)])

    return phase_a


def _make_phase_b(E, D, NP):
    ept = E // (NC * NS)
    n_chunks = ept // CH
    rpt = NP // NS

    @functools.partial(
        pl.kernel,
        out_type=jax.ShapeDtypeStruct((E, D), jnp.float32),
        mesh=_mesh(),
        scratch_types=[
            pltpu.VMEM_SHARED((NP, D), jnp.float32),
            pltpu.VMEM((2, CR, D), jnp.float32),
            pltpu.VMEM((2, CR, D), jnp.float32),
            pltpu.VMEM((2 * CR,), jnp.float32),
            pltpu.VMEM((2 * CR,), jnp.float32),
            pltpu.VMEM((ept,), jnp.int32),
            pltpu.VMEM((2, CH, D), jnp.float32),
            pltpu.SemaphoreType.DMA,
            pltpu.SemaphoreType.DMA((4,)),
            pltpu.SemaphoreType.DMA((2,)),
        ],
    )
    def phase_b(acc_hbm, cnt_hbm, seg_hbm, out_hbm, table_sh,
                a0_v, a1_v, c0_v, c1_v, idx_v, rows_v,
                sem_ix, sem_cl, sem_w):
        c = lax.axis_index("c")
        s = lax.axis_index("s")
        g = c * NS + s

        idx_ld = pltpu.async_copy(
            seg_hbm.at[pl.ds(g * ept, ept)], idx_v, sem_ix)

        def issue_cloads(j, m):
            rb = s * rpt + j * CR
            pltpu.async_copy(acc_hbm.at[0, pl.ds(rb, CR), :], a0_v.at[m], sem_cl.at[0])
            pltpu.async_copy(acc_hbm.at[1, pl.ds(rb, CR), :], a1_v.at[m], sem_cl.at[1])
            pltpu.async_copy(
                cnt_hbm.at[0, pl.ds(rb, CR)], c0_v.at[pl.ds(m * CR, CR)], sem_cl.at[2])
            pltpu.async_copy(
                cnt_hbm.at[1, pl.ds(rb, CR)], c1_v.at[pl.ds(m * CR, CR)], sem_cl.at[3])

        def wait_cloads(j, m):
            rb = s * rpt + j * CR
            pltpu.make_async_copy(
                acc_hbm.at[0, pl.ds(rb, CR), :], a0_v.at[m], sem_cl.at[0]).wait()
            pltpu.make_async_copy(
                acc_hbm.at[1, pl.ds(rb, CR), :], a1_v.at[m], sem_cl.at[1]).wait()
            pltpu.make_async_copy(
                cnt_hbm.at[0, pl.ds(rb, CR)],
                c0_v.at[pl.ds(m * CR, CR)], sem_cl.at[2]).wait()
            pltpu.make_async_copy(
                cnt_hbm.at[1, pl.ds(rb, CR)],
                c1_v.at[pl.ds(m * CR, CR)], sem_cl.at[3]).wait()

        def combine(j, m, prefetch):
            wait_cloads(j, m)
            if prefetch:
                issue_cloads(j + 1, 1 - m)

            def grp_body(gi, _):
                base = gi * L
                gsl = pl.ds(base, L)
                msl = pl.ds(m * CR + base, L)
                rcp16 = 1.0 / (c0_v[msl] + c1_v[msl] + jnp.float32(EPS))
                for jj in range(L):
                    rcp = rcp16[jj]
                    r = base + jj
                    for k in range(D // L):
                        sl = pl.ds(k * L, L)
                        a0_v[m, r, sl] = (a0_v[m, r, sl] + a1_v[m, r, sl]) * rcp
                return 0

            lax.fori_loop(0, CR // L, grp_body, 0)
            rb = s * rpt + j * CR
            pltpu.sync_copy(a0_v.at[m], table_sh.at[pl.ds(rb, CR), :])

        n_comb = rpt // CR
        assert n_comb % 2 == 0
        issue_cloads(0, 0)
        combine(0, 0, True)

        def comb_pair(p, _):
            combine(2 * p + 1, 1, True)
            combine(2 * p + 2, 0, True)
            return 0

        lax.fori_loop(0, (n_comb - 2) // 2, comb_pair, 0)
        combine(n_comb - 1, 1, False)
        idx_ld.wait()
        plsc.subcore_barrier()

        def chunk(j, b, drain):
            if drain:
                pltpu.make_async_copy(
                    rows_v.at[b], out_hbm.at[pl.ds(0, CH), :], sem_w.at[b]).wait()
            eb = g * ept + j * CH
            pltpu.sync_copy(table_sh.at[idx_v.at[pl.ds(j * CH, CH)]], rows_v.at[b])
            pltpu.async_copy(
                rows_v.at[b], out_hbm.at[pl.ds(eb, CH), :], sem_w.at[b])

        chunk(0, 0, False)
        chunk(1, 1, False)

        def pair_body(p, _):
            chunk(2 * p, 0, True)
            chunk(2 * p + 1, 1, True)
            return 0

        lax.fori_loop(1, n_chunks // 2, pair_body, 0)
        if n_chunks % 2 == 1:
            chunk(n_chunks - 1, 0, True)
        for b in range(2):
            pltpu.make_async_copy(
                rows_v.at[b], out_hbm.at[pl.ds(0, CH), :], sem_w.at[b]).wait()

    return phase_b


def kernel(input, index):
    E, D = input.shape
    NP = ((10000 + NS * CR - 1) // (NS * CR)) * (NS * CR)
    seg = index[:, 1]
    seg3 = seg.reshape(NC * NS, (E // CH) // (NC * NS), CH)
    acc, cnt = _make_phase_a(E, D, NP)(input, seg3)
    return _make_phase_b(E, D, NP)(acc, cnt, seg)


# speedup vs baseline: 8.1311x; 1.0181x over previous
import functools

import jax
import jax.numpy as jnp
from jax import lax
from jax.experimental import pallas as pl
from jax.experimental.pallas import tpu as pltpu
from jax.experimental.pallas import tpu_sc as plsc

NC = 2
NS = 16
L = 16
EPS = 1e-9

CH = 80
CR = 16
CZ = 16


def _mesh():
    return plsc.VectorSubcoreMesh(
        core_axis_name="c", subcore_axis_name="s", num_cores=NC, num_subcores=NS
    )


def _make_phase_a(E, D, NP):
    ept = E // (NC * NS)
    n_chunks = ept // CH
    rpt = NP // NS

    @functools.partial(
        pl.kernel,
        out_type=[
            jax.ShapeDtypeStruct((NC, NP, D), jnp.float32),
            jax.ShapeDtypeStruct((NC, NP), jnp.float32),
        ],
        mesh=_mesh(),
        scratch_types=[
            pltpu.VMEM_SHARED((NP, D), jnp.float32),
            pltpu.VMEM_SHARED((NP,), jnp.float32),
            pltpu.VMEM((2, CH, D), jnp.float32),
            pltpu.VMEM((n_chunks, CH), jnp.int32),
            pltpu.VMEM((CH,), jnp.float32),
            pltpu.VMEM((CZ, D), jnp.float32),
            pltpu.VMEM((rpt,), jnp.float32),
            pltpu.SemaphoreType.DMA,
            pltpu.SemaphoreType.DMA((2,)),
            pltpu.SemaphoreType.DMA((2,)),
            pltpu.SemaphoreType.DMA((2,)),
        ],
    )
    def phase_a(x_hbm, seg_hbm, acc_out, cnt_out, acc_sh, cnt_sh,
                rows_v, idx_v, ones_v, zrow_v, zcnt_v,
                sem_ix, sem_lr, sem_sa, sem_sc):
        c = lax.axis_index("c")
        s = lax.axis_index("s")
        g = c * NS + s

        idx_ld = pltpu.async_copy(seg_hbm.at[g], idx_v, sem_ix)

        zero = jnp.zeros((L,), jnp.float32)
        one = jnp.ones((L,), jnp.float32)
        for i in range(CH // L):
            ones_v[pl.ds(i * L, L)] = one
        for i in range(rpt // L):
            zcnt_v[pl.ds(i * L, L)] = zero

        def zrow_body(r, _):
            for k in range(D // L):
                zrow_v[r, pl.ds(k * L, L)] = zero
            return 0

        lax.fori_loop(0, CZ, zrow_body, 0)

        def zinit_body(j, _):
            rb = s * rpt + j * CZ
            pltpu.sync_copy(zrow_v, acc_sh.at[pl.ds(rb, CZ), :])
            return 0

        lax.fori_loop(0, rpt // CZ, zinit_body, 0)
        pltpu.sync_copy(zcnt_v, cnt_sh.at[pl.ds(s * rpt, rpt)])
        idx_ld.wait()
        plsc.subcore_barrier()

        def chunk(j, b, jd):
            if jd is not None:
                pltpu.make_async_copy(
                    rows_v.at[b], acc_sh.at[idx_v.at[jd]], sem_sa.at[b]).wait()
                pltpu.make_async_copy(
                    ones_v, cnt_sh.at[idx_v.at[jd]], sem_sc.at[b]).wait()
            eb = g * ept + j * CH
            pltpu.async_copy(
                x_hbm.at[pl.ds(eb, CH), :], rows_v.at[b], sem_lr.at[b]).wait()
            pltpu.async_copy(
                rows_v.at[b], acc_sh.at[idx_v.at[j]], sem_sa.at[b], add=True)
            pltpu.async_copy(
                ones_v, cnt_sh.at[idx_v.at[j]], sem_sc.at[b], add=True)

        chunk(0, 0, None)
        chunk(1, 1, None)

        def pair_body(p, _):
            j = 2 * p
            chunk(j, 0, j - 2)
            chunk(j + 1, 1, j - 1)
            return 0

        lax.fori_loop(1, n_chunks // 2, pair_body, 0)
        assert n_chunks % 2 == 1
        chunk(n_chunks - 1, 0, n_chunks - 3)
        for b, jd in ((0, n_chunks - 1), (1, n_chunks - 2)):
            pltpu.make_async_copy(
                rows_v.at[b], acc_sh.at[idx_v.at[jd]], sem_sa.at[b]).wait()
            pltpu.make_async_copy(
                ones_v, cnt_sh.at[idx_v.at[jd]], sem_sc.at[b]).wait()
        plsc.subcore_barrier()

        rb = s * rpt
        pltpu.sync_copy(acc_sh.at[pl.ds(rb, rpt), :], acc_out.at[c, pl.ds(rb, rpt), :])
        pltpu.sync_copy(cnt_sh.at[pl.ds(rb, rpt)], cnt_out.at[c, pl.ds(rb, rpt)])

    return phase_a


def _make_phase_b(E, D, NP):
    ept = E // (NC * NS)
    n_chunks = ept // CH
    rpt = NP // NS

    @functools.partial(
        pl.kernel,
        out_type=jax.ShapeDtypeStruct((E, D), jnp.float32),
        mesh=_mesh(),
        scratch_types=[
            pltpu.VMEM_SHARED((NP, D), jnp.float32),
            pltpu.VMEM((2, CR, D), jnp.float32),
            pltpu.VMEM((2, CR, D), jnp.float32),
            pltpu.VMEM((2 * CR,), jnp.float32),
            pltpu.VMEM((2 * CR,), jnp.float32),
            pltpu.VMEM((ept,), jnp.int32),
            pltpu.VMEM((2, CH, D), jnp.float32),
            pltpu.SemaphoreType.DMA,
            pltpu.SemaphoreType.DMA((4,)),
            pltpu.SemaphoreType.DMA((2,)),
            pltpu.SemaphoreType.DMA((2,)),
        ],
    )
    def phase_b(acc_hbm, cnt_hbm, seg_hbm, out_hbm, table_sh,
                a0_v, a1_v, c0_v, c1_v, idx_v, rows_v,
                sem_ix, sem_cl, sem_g, sem_w):
        c = lax.axis_index("c")
        s = lax.axis_index("s")
        g = c * NS + s

        idx_ld = pltpu.async_copy(
            seg_hbm.at[pl.ds(g * ept, ept)], idx_v, sem_ix)

        def issue_cloads(j, m):
            rb = s * rpt + j * CR
            pltpu.async_copy(acc_hbm.at[0, pl.ds(rb, CR), :], a0_v.at[m], sem_cl.at[0])
            pltpu.async_copy(acc_hbm.at[1, pl.ds(rb, CR), :], a1_v.at[m], sem_cl.at[1])
            pltpu.async_copy(
                cnt_hbm.at[0, pl.ds(rb, CR)], c0_v.at[pl.ds(m * CR, CR)], sem_cl.at[2])
            pltpu.async_copy(
                cnt_hbm.at[1, pl.ds(rb, CR)], c1_v.at[pl.ds(m * CR, CR)], sem_cl.at[3])

        def wait_cloads(j, m):
            rb = s * rpt + j * CR
            pltpu.make_async_copy(
                acc_hbm.at[0, pl.ds(rb, CR), :], a0_v.at[m], sem_cl.at[0]).wait()
            pltpu.make_async_copy(
                acc_hbm.at[1, pl.ds(rb, CR), :], a1_v.at[m], sem_cl.at[1]).wait()
            pltpu.make_async_copy(
                cnt_hbm.at[0, pl.ds(rb, CR)],
                c0_v.at[pl.ds(m * CR, CR)], sem_cl.at[2]).wait()
            pltpu.make_async_copy(
                cnt_hbm.at[1, pl.ds(rb, CR)],
                c1_v.at[pl.ds(m * CR, CR)], sem_cl.at[3]).wait()

        def combine(j, m, prefetch):
            wait_cloads(j, m)
            if prefetch:
                issue_cloads(j + 1, 1 - m)

            def grp_body(gi, _):
                base = gi * L
                gsl = pl.ds(base, L)
                msl = pl.ds(m * CR + base, L)
                rcp16 = 1.0 / (c0_v[msl] + c1_v[msl] + jnp.float32(EPS))
                for jj in range(L):
                    rcp = rcp16[jj]
                    r = base + jj
                    for k in range(D // L):
                        sl = pl.ds(k * L, L)
                        a0_v[m, r, sl] = (a0_v[m, r, sl] + a1_v[m, r, sl]) * rcp
                return 0

            lax.fori_loop(0, CR // L, grp_body, 0)
            rb = s * rpt + j * CR
            pltpu.sync_copy(a0_v.at[m], table_sh.at[pl.ds(rb, CR), :])

        n_comb = rpt // CR
        assert n_comb % 2 == 0
        issue_cloads(0, 0)
        combine(0, 0, True)

        def comb_pair(p, _):
            combine(2 * p + 1, 1, True)
            combine(2 * p + 2, 0, True)
            return 0

        lax.fori_loop(0, (n_comb - 2) // 2, comb_pair, 0)
        combine(n_comb - 1, 1, False)
        idx_ld.wait()
        plsc.subcore_barrier()

        def issue_gather(j, b):
            pltpu.async_copy(
                table_sh.at[idx_v.at[pl.ds(j * CH, CH)]], rows_v.at[b],
                sem_g.at[b])

        def wait_gather(j, b):
            pltpu.make_async_copy(
                table_sh.at[idx_v.at[pl.ds(j * CH, CH)]], rows_v.at[b],
                sem_g.at[b]).wait()

        def issue_write(j, b):
            eb = g * ept + j * CH
            pltpu.async_copy(
                rows_v.at[b], out_hbm.at[pl.ds(eb, CH), :], sem_w.at[b])

        def drain_write(b):
            pltpu.make_async_copy(
                rows_v.at[b], out_hbm.at[pl.ds(0, CH), :], sem_w.at[b]).wait()

        def visit(j, b, drain):
            if drain:
                drain_write(b)
            issue_gather(j, b)
            wait_gather(j - 1, 1 - b)
            issue_write(j - 1, 1 - b)

        issue_gather(0, 0)
        visit(1, 1, False)

        def pair_body(p, _):
            visit(2 * p, 0, True)
            visit(2 * p + 1, 1, True)
            return 0

        lax.fori_loop(1, n_chunks // 2, pair_body, 0)
        assert n_chunks % 2 == 1
        visit(n_chunks - 1, 0, True)
        wait_gather(n_chunks - 1, 0)
        issue_write(n_chunks - 1, 0)
        drain_write(1)
        drain_write(0)

    return phase_b


def kernel(input, index):
    E, D = input.shape
    NP = ((10000 + NS * CR - 1) // (NS * CR)) * (NS * CR)
    seg = index[:, 1]
    seg3 = seg.reshape(NC * NS, (E // CH) // (NC * NS), CH)
    acc, cnt = _make_phase_a(E, D, NP)(input, seg3)
    return _make_phase_b(E, D, NP)(acc, cnt, seg)
